# Initial kernel scaffold; baseline (speedup 1.0000x reference)
import jax
import jax.numpy as jnp
from jax.experimental import pallas as pl


def kernel(edges1, edges2, num_nodes):
    raise NotImplementedError("write your pallas kernel here")


# speedup vs baseline: 1.0958x; 1.0958x over previous
import functools

import jax
import jax.numpy as jnp
from jax import lax
from jax.experimental import pallas as pl
from jax.experimental.pallas import tpu as pltpu
from jax.experimental.pallas import tpu_sc as plsc

_N = 10000
_E = 160000
_NW = 32
_CHUNK = _E // _NW
_RW = 128
_NR = 40
_CP = _NR * _RW
_P = _NW * _CP
_PADKEY = _N * _N
_TBL = _PADKEY + 64

_mesh = plsc.VectorSubcoreMesh(core_axis_name="c", subcore_axis_name="s")
_i32 = jnp.int32


def _wid():
    return lax.axis_index("s") * 2 + lax.axis_index("c")


def _scatter_body(e1r, e1c, e2r, e2c,
                  k1p, k1q, k2p, k2q, tab_a, tab_b,
                  rbuf, cbuf, klin, kq, vbuf, sem):
    wid = _wid()
    base = wid * _CHUNK
    pbase = wid * _CP
    iota = lax.iota(_i32, 16)
    for er, ec, kp_out, kq_out, tab in (
            (e1r, e1c, k1p, k1q, tab_a),
            (e2r, e2c, k2p, k2q, tab_b)):
        pltpu.sync_copy(er.at[pl.ds(base, _CHUNK)], rbuf.at[pl.ds(0, _CHUNK)])
        pltpu.sync_copy(ec.at[pl.ds(base, _CHUNK)], cbuf.at[pl.ds(0, _CHUNK)])

        def row(r, _):
            for u in range(8):
                off = r * _RW + u * 16
                lanes = off + iota
                rv = rbuf[pl.ds(off, 16)]
                cv = cbuf[pl.ds(off, 16)]
                rv = jnp.minimum(jnp.maximum(rv, 0), _N - 1)
                cv = jnp.minimum(jnp.maximum(cv, 0), _N - 1)
                key = jnp.where(lanes < _CHUNK, rv * _N + cv, _PADKEY)
                klin[pl.ds(off, 16)] = key
                kq[r, pl.ds(u * 16, 16)] = key
                vbuf[r, pl.ds(u * 16, 16)] = pbase + lanes
            return 0

        lax.fori_loop(0, _NR, row, 0)
        pltpu.sync_copy(klin, kp_out.at[pl.ds(pbase, _CP)])
        pltpu.sync_copy(kq, kq_out.at[wid])

        def scat(r, _):
            pltpu.async_copy(vbuf.at[r], tab.at[kq.at[r]], sem)
            return 0

        lax.fori_loop(0, _NR, scat, 0)
        pltpu.make_async_copy(kq_out.at[wid], vbuf, sem).wait()


def _count_body(k1p, k1q, k2p, k2q, tab_a, tab_b,
                parts, kb1, kb2, g1, g2, g3, vb, gk, res, sem):
    wid = _wid()
    pbase = wid * _CP
    iota = lax.iota(_i32, 16)
    pltpu.sync_copy(k1q.at[wid], kb1)
    pltpu.sync_copy(k2q.at[wid], kb2)
    def gat(r, _):
        pltpu.async_copy(tab_a.at[kb1.at[r]], g1.at[r], sem)
        pltpu.async_copy(tab_b.at[kb2.at[r]], g2.at[r], sem)
        pltpu.async_copy(tab_a.at[kb2.at[r]], g3.at[r], sem)
        return 0

    lax.fori_loop(0, _NR, gat, 0)
    pltpu.make_async_copy(k1q.at[wid], g1, sem).wait()
    pltpu.make_async_copy(k1q.at[wid], g2, sem).wait()
    pltpu.make_async_copy(k1q.at[wid], g3, sem).wait()

    def rowv(r, _):
        for u in range(8):
            v = g3[r, pl.ds(u * 16, 16)]
            vb[r, pl.ds(u * 16, 16)] = jnp.minimum(jnp.maximum(v, 0), _P - 1)
        return 0

    lax.fori_loop(0, _NR, rowv, 0)

    def gat2(r, _):
        pltpu.async_copy(k1p.at[vb.at[r]], gk.at[r], sem)
        return 0

    lax.fori_loop(0, _NR, gat2, 0)
    pltpu.make_async_copy(k1q.at[wid], gk, sem).wait()

    def rowa(r, carry):
        a1, a2, ai = carry
        for u in range(8):
            off = r * _RW + u * 16
            lanes = off + iota
            valid = lanes < _CHUNK
            pos = pbase + lanes
            x1 = g1[r, pl.ds(u * 16, 16)]
            x2 = g2[r, pl.ds(u * 16, 16)]
            xk = gk[r, pl.ds(u * 16, 16)]
            k2v = kb2[r, pl.ds(u * 16, 16)]
            one = jnp.ones((16,), _i32)
            zero = jnp.zeros((16,), _i32)
            a1 = a1 + jnp.where(valid & (x1 == pos), one, zero)
            w2 = valid & (x2 == pos)
            a2 = a2 + jnp.where(w2, one, zero)
            ai = ai + jnp.where(w2 & (xk == k2v), one, zero)
        return a1, a2, ai

    z = jnp.zeros((16,), _i32)
    a1, a2, ai = lax.fori_loop(0, _NR, rowa, (z, z, z))
    res[pl.ds(0, 16)] = a1
    res[pl.ds(16, 16)] = a2
    res[pl.ds(32, 16)] = ai
    pltpu.sync_copy(res, parts.at[wid])


_scatter_call = pl.kernel(
    _scatter_body,
    out_type=[
        jax.ShapeDtypeStruct((_P,), _i32),
        jax.ShapeDtypeStruct((_NW, _NR, _RW), _i32),
        jax.ShapeDtypeStruct((_P,), _i32),
        jax.ShapeDtypeStruct((_NW, _NR, _RW), _i32),
        jax.ShapeDtypeStruct((_TBL,), _i32),
        jax.ShapeDtypeStruct((_TBL,), _i32),
    ],
    mesh=_mesh,
    scratch_types=[
        pltpu.VMEM((_CP,), _i32),
        pltpu.VMEM((_CP,), _i32),
        pltpu.VMEM((_CP,), _i32),
        pltpu.VMEM((_NR, _RW), _i32),
        pltpu.VMEM((_NR, _RW), _i32),
        pltpu.SemaphoreType.DMA,
    ],
)

_count_call = pl.kernel(
    _count_body,
    out_type=[jax.ShapeDtypeStruct((_NW, 48), _i32)],
    mesh=_mesh,
    scratch_types=[
        pltpu.VMEM((_NR, _RW), _i32),
        pltpu.VMEM((_NR, _RW), _i32),
        pltpu.VMEM((_NR, _RW), _i32),
        pltpu.VMEM((_NR, _RW), _i32),
        pltpu.VMEM((_NR, _RW), _i32),
        pltpu.VMEM((_NR, _RW), _i32),
        pltpu.VMEM((_NR, _RW), _i32),
        pltpu.VMEM((48,), _i32),
        pltpu.SemaphoreType.DMA,
    ],
)


def kernel(edges1, edges2, num_nodes):
    del num_nodes
    e1 = edges1.astype(_i32)
    e2 = edges2.astype(_i32)
    k1p, k1q, k2p, k2q, tab_a, tab_b = _scatter_call(
        e1[0], e1[1], e2[0], e2[1])
    (parts,) = _count_call(k1p, k1q, k2p, k2q, tab_a, tab_b)
    d1 = parts[:, 0:16].sum()
    d2 = parts[:, 16:32].sum()
    it = parts[:, 32:48].sum()
    union = d1 + d2 - it
    return it.astype(jnp.float32) / union.astype(jnp.float32)


# speedup vs baseline: 1.1219x; 1.0239x over previous
import jax
import jax.numpy as jnp
from jax import lax
from jax.experimental import pallas as pl
from jax.experimental.pallas import tpu as pltpu
from jax.experimental.pallas import tpu_sc as plsc

_N = 10000
_E = 160000
_NW = 32
_CHUNK = _E // _NW
_CP = 5120
_NV = _CP // 16
_P = _NW * _CP
_PADKEY = _N * _N
_TBL = _PADKEY + 64

_mesh = plsc.VectorSubcoreMesh(core_axis_name="c", subcore_axis_name="s")
_i32 = jnp.int32


def _wid():
    return lax.axis_index("s") * 2 + lax.axis_index("c")


def _scatter_body(e1r, e1c, e2r, e2c,
                  k1p, k2p, tab_a, tab_b,
                  rbuf, cbuf, k1b, v1b, k2b, v2b, sem):
    wid = _wid()
    iota = lax.iota(_i32, 16)
    pbase = wid * _CP
    for er, ec, kp_out, kb, vb, tab in (
            (e1r, e1c, k1p, k1b, v1b, tab_a),
            (e2r, e2c, k2p, k2b, v2b, tab_b)):
        base = wid * _CHUNK
        pltpu.sync_copy(er.at[pl.ds(base, _CHUNK)], rbuf.at[pl.ds(0, _CHUNK)])
        pltpu.sync_copy(ec.at[pl.ds(base, _CHUNK)], cbuf.at[pl.ds(0, _CHUNK)])

        def step(m, _):
            off = m * 16
            lanes = off + iota
            rv = rbuf[pl.ds(off, 16)]
            cv = cbuf[pl.ds(off, 16)]
            rv = jnp.minimum(jnp.maximum(rv, 0), _N - 1)
            cv = jnp.minimum(jnp.maximum(cv, 0), _N - 1)
            key = jnp.where(lanes < _CHUNK, rv * _N + cv, _PADKEY)
            kb[pl.ds(off, 16)] = key
            vb[pl.ds(off, 16)] = pbase + lanes
            return 0

        lax.fori_loop(0, _NV, step, 0)
        pltpu.sync_copy(kb, kp_out.at[pl.ds(pbase, _CP)])
        pltpu.async_copy(vb, tab.at[kb], sem)
    pltpu.make_async_copy(k1p.at[pl.ds(pbase, _CP)], v1b, sem).wait()
    pltpu.make_async_copy(k1p.at[pl.ds(pbase, _CP)], v2b, sem).wait()


def _count_body(k1p, k2p, tab_a, tab_b,
                parts, kb1, kb2, g1, g2, g3, gk, res, sem):
    wid = _wid()
    pbase = wid * _CP
    iota = lax.iota(_i32, 16)
    pltpu.sync_copy(k1p.at[pl.ds(pbase, _CP)], kb1)
    pltpu.sync_copy(k2p.at[pl.ds(pbase, _CP)], kb2)
    pltpu.async_copy(tab_a.at[kb1], g1, sem)
    pltpu.async_copy(tab_b.at[kb2], g2, sem)
    pltpu.async_copy(tab_a.at[kb2], g3, sem)
    dummy = k1p.at[pl.ds(pbase, _CP)]
    pltpu.make_async_copy(dummy, g1, sem).wait()
    pltpu.make_async_copy(dummy, g2, sem).wait()
    pltpu.make_async_copy(dummy, g3, sem).wait()

    def rowv(m, _):
        off = m * 16
        v = g3[pl.ds(off, 16)]
        g3[pl.ds(off, 16)] = jnp.minimum(jnp.maximum(v, 0), _P - 1)
        return 0

    lax.fori_loop(0, _NV, rowv, 0)
    pltpu.async_copy(k1p.at[g3], gk, sem)
    pltpu.make_async_copy(dummy, gk, sem).wait()

    def rowa(m, carry):
        a1, a2, ai = carry
        off = m * 16
        lanes = off + iota
        valid = lanes < _CHUNK
        pos = pbase + lanes
        x1 = g1[pl.ds(off, 16)]
        x2 = g2[pl.ds(off, 16)]
        xk = gk[pl.ds(off, 16)]
        k2v = kb2[pl.ds(off, 16)]
        one = jnp.ones((16,), _i32)
        zero = jnp.zeros((16,), _i32)
        a1 = a1 + jnp.where(valid & (x1 == pos), one, zero)
        w2 = valid & (x2 == pos)
        a2 = a2 + jnp.where(w2, one, zero)
        ai = ai + jnp.where(w2 & (xk == k2v), one, zero)
        return a1, a2, ai

    z = jnp.zeros((16,), _i32)
    a1, a2, ai = lax.fori_loop(0, _NV, rowa, (z, z, z))
    res[pl.ds(0, 16)] = a1
    res[pl.ds(16, 16)] = a2
    res[pl.ds(32, 16)] = ai
    pltpu.sync_copy(res, parts.at[wid])


_scatter_call = pl.kernel(
    _scatter_body,
    out_type=[
        jax.ShapeDtypeStruct((_P,), _i32),
        jax.ShapeDtypeStruct((_P,), _i32),
        jax.ShapeDtypeStruct((_TBL,), _i32),
        jax.ShapeDtypeStruct((_TBL,), _i32),
    ],
    mesh=_mesh,
    scratch_types=[
        pltpu.VMEM((_CP,), _i32),
        pltpu.VMEM((_CP,), _i32),
        pltpu.VMEM((_CP,), _i32),
        pltpu.VMEM((_CP,), _i32),
        pltpu.VMEM((_CP,), _i32),
        pltpu.VMEM((_CP,), _i32),
        pltpu.SemaphoreType.DMA,
    ],
)

_count_call = pl.kernel(
    _count_body,
    out_type=[jax.ShapeDtypeStruct((_NW, 48), _i32)],
    mesh=_mesh,
    scratch_types=[
        pltpu.VMEM((_CP,), _i32),
        pltpu.VMEM((_CP,), _i32),
        pltpu.VMEM((_CP,), _i32),
        pltpu.VMEM((_CP,), _i32),
        pltpu.VMEM((_CP,), _i32),
        pltpu.VMEM((_CP,), _i32),
        pltpu.VMEM((48,), _i32),
        pltpu.SemaphoreType.DMA,
    ],
)


def kernel(edges1, edges2, num_nodes):
    del num_nodes
    e1 = edges1.astype(_i32)
    e2 = edges2.astype(_i32)
    k1p, k2p, tab_a, tab_b = _scatter_call(e1[0], e1[1], e2[0], e2[1])
    (parts,) = _count_call(k1p, k2p, tab_a, tab_b)
    d1 = parts[:, 0:16].sum()
    d2 = parts[:, 16:32].sum()
    it = parts[:, 32:48].sum()
    union = d1 + d2 - it
    return it.astype(jnp.float32) / union.astype(jnp.float32)
